# baseline (speedup 1.0000x reference)
import jax
import jax.numpy as jnp
from jax import lax
from jax.experimental import pallas as pl

_THRESH = 0.6


def _fused_kernel(data_ref, wg_ref, bg_ref, wc_ref, bc_ref, out_ref):
    cb = data_ref[...]
    t = jnp.float32(data_ref.shape[1])
    c2b = cb * cb
    ones = jnp.ones((data_ref.shape[1], 8), dtype=jnp.bfloat16)

    g = lax.dot_general(
        cb, cb, (((1,), (1,)), ((), ())), preferred_element_type=jnp.float32)
    s1 = jnp.dot(cb, ones, preferred_element_type=jnp.float32)[:, 0:1]
    s3 = jnp.dot(c2b * cb, ones, preferred_element_type=jnp.float32)[:, 0:1]
    s4 = jnp.dot(c2b * c2b, ones, preferred_element_type=jnp.float32)[:, 0:1]

    row = lax.broadcasted_iota(jnp.int32, g.shape, 0)
    colid = lax.broadcasted_iota(jnp.int32, g.shape, 1)
    eye = row == colid

    s2 = jnp.sum(jnp.where(eye, g, 0.0), axis=1, keepdims=True)
    mu = s1 / t
    ex2 = s2 / t
    ex3 = s3 / t
    ex4 = s4 / t
    mu2 = mu * mu
    m2 = ex2 - mu2
    m3 = ex3 - 3.0 * mu * ex2 + 2.0 * mu * mu2
    m4 = ex4 - 4.0 * mu * ex3 + 6.0 * mu2 * ex2 - 3.0 * mu2 * mu2
    m2s = jnp.maximum(m2, 1e-12)
    skew = m3 / (m2s * jnp.sqrt(m2s))
    kurt = m4 / (m2s * m2s) - 3.0
    x = jnp.concatenate([mu, m2, skew, kurt], axis=1)

    outer = lax.dot_general(
        s1, s1, (((1,), (1,)), ((), ())), preferred_element_type=jnp.float32)
    cov = g - outer / t
    normsq = jnp.maximum(s2 - s1 * s1 / t, 0.0)
    norm = jnp.maximum(jnp.sqrt(normsq), 1e-6)
    denom = lax.dot_general(
        norm, norm, (((1,), (1,)), ((), ())),
        preferred_element_type=jnp.float32)
    corr = jnp.clip(cov / denom, -1.0, 1.0)

    acorr = jnp.abs(corr)
    w = jnp.clip(acorr, 1e-6, 0.99)
    a_mat = jnp.where(eye, 1.0, jnp.where(acorr >= _THRESH, w, 0.0))

    agg = jnp.dot(a_mat, x, preferred_element_type=jnp.float32)
    h = jnp.maximum(
        jnp.dot(agg, wg_ref[...], preferred_element_type=jnp.float32)
        + bg_ref[...], 0.0)
    emb = jnp.sum(h, axis=0, keepdims=True)
    logits = jnp.dot(emb, wc_ref[...],
                     preferred_element_type=jnp.float32) + bc_ref[...]
    out_ref[...] = logits


def kernel(data, W_gfc, b_gfc, W_cls, b_cls):
    c_rows, t_total = data.shape
    hidden = W_gfc.shape[1]
    n_cls = W_cls.shape[1]

    bg = b_gfc.reshape(1, hidden)
    bc = b_cls.reshape(1, n_cls)
    db = data.astype(jnp.bfloat16)

    out = pl.pallas_call(
        _fused_kernel,
        out_shape=jax.ShapeDtypeStruct((1, n_cls), jnp.float32),
    )(db, W_gfc, bg, W_cls, bc)
    return out


# speedup vs baseline: 1.0140x; 1.0140x over previous
import functools

import jax
import jax.numpy as jnp
from jax import lax
from jax.experimental import pallas as pl
from jax.experimental.pallas import tpu as pltpu

_THRESH = 0.6


def _fused_kernel(data_ref, wg_ref, bg_ref, wc_ref, bc_ref, out_ref,
                  g_ref, s1_ref, s3_ref, s4_ref,
                  *, t_total, chunk, n_chunks):
    step = pl.program_id(0)

    @pl.when(step == 0)
    def _init():
        g_ref[...] = jnp.zeros_like(g_ref)
        s1_ref[...] = jnp.zeros_like(s1_ref)
        s3_ref[...] = jnp.zeros_like(s3_ref)
        s4_ref[...] = jnp.zeros_like(s4_ref)

    c_rows = data_ref.shape[0]

    def _accum(cb):
        c2b = cb * cb
        ones = jnp.ones((chunk, 8), dtype=jnp.bfloat16)
        g_ref[...] += lax.dot_general(
            cb, cb, (((1,), (1,)), ((), ())),
            preferred_element_type=jnp.float32)
        s1_ref[...] += jnp.dot(cb, ones, preferred_element_type=jnp.float32)
        s3_ref[...] += jnp.dot(c2b * cb, ones,
                               preferred_element_type=jnp.float32)
        s4_ref[...] += jnp.dot(c2b * c2b, ones,
                               preferred_element_type=jnp.float32)

    @pl.when(step < n_chunks - 1)
    def _full_chunk():
        _accum(data_ref[...])

    @pl.when(step == n_chunks - 1)
    def _last_chunk():
        col = lax.broadcasted_iota(jnp.int32, (c_rows, chunk), 1)
        valid = col < (t_total - (n_chunks - 1) * chunk)
        _accum(jnp.where(valid, data_ref[...], jnp.bfloat16(0.0)))

    @pl.when(step == n_chunks - 1)
    def _finalize():
        t = jnp.float32(t_total)
        g = g_ref[...]
        row = lax.broadcasted_iota(jnp.int32, g.shape, 0)
        colid = lax.broadcasted_iota(jnp.int32, g.shape, 1)
        eye = row == colid

        s1 = s1_ref[:, 0:1]
        s2 = jnp.sum(jnp.where(eye, g, 0.0), axis=1, keepdims=True)
        s3 = s3_ref[:, 0:1]
        s4 = s4_ref[:, 0:1]
        mu = s1 / t
        ex2 = s2 / t
        ex3 = s3 / t
        ex4 = s4 / t
        mu2 = mu * mu
        m2 = ex2 - mu2
        m3 = ex3 - 3.0 * mu * ex2 + 2.0 * mu * mu2
        m4 = ex4 - 4.0 * mu * ex3 + 6.0 * mu2 * ex2 - 3.0 * mu2 * mu2
        m2s = jnp.maximum(m2, 1e-12)
        skew = m3 / (m2s * jnp.sqrt(m2s))
        kurt = m4 / (m2s * m2s) - 3.0
        x = jnp.concatenate([mu, m2, skew, kurt], axis=1)

        outer = lax.dot_general(
            s1, s1, (((1,), (1,)), ((), ())),
            preferred_element_type=jnp.float32)
        cov = g - outer / t
        normsq = jnp.maximum(s2 - s1 * s1 / t, 0.0)
        norm = jnp.maximum(jnp.sqrt(normsq), 1e-6)
        denom = lax.dot_general(
            norm, norm, (((1,), (1,)), ((), ())),
            preferred_element_type=jnp.float32)
        corr = jnp.clip(cov / denom, -1.0, 1.0)

        acorr = jnp.abs(corr)
        w = jnp.clip(acorr, 1e-6, 0.99)
        a_mat = jnp.where(eye, 1.0, jnp.where(acorr >= _THRESH, w, 0.0))

        agg = jnp.dot(a_mat, x, preferred_element_type=jnp.float32)
        h = jnp.maximum(
            jnp.dot(agg, wg_ref[...], preferred_element_type=jnp.float32)
            + bg_ref[...], 0.0)
        emb = jnp.sum(h, axis=0, keepdims=True)
        logits = jnp.dot(emb, wc_ref[...],
                         preferred_element_type=jnp.float32) + bc_ref[...]
        out_ref[...] = logits


def kernel(data, W_gfc, b_gfc, W_cls, b_cls):
    c_rows, t_total = data.shape
    hidden = W_gfc.shape[1]
    n_cls = W_cls.shape[1]
    chunk = 2560
    n_chunks = (t_total + chunk - 1) // chunk

    bg = b_gfc.reshape(1, hidden)
    bc = b_cls.reshape(1, n_cls)
    db = data.astype(jnp.bfloat16)

    body = functools.partial(
        _fused_kernel, t_total=t_total, chunk=chunk, n_chunks=n_chunks)
    out = pl.pallas_call(
        body,
        grid=(n_chunks,),
        in_specs=[
            pl.BlockSpec((c_rows, chunk), lambda i: (0, i)),
            pl.BlockSpec(W_gfc.shape, lambda i: (0, 0)),
            pl.BlockSpec(bg.shape, lambda i: (0, 0)),
            pl.BlockSpec(W_cls.shape, lambda i: (0, 0)),
            pl.BlockSpec(bc.shape, lambda i: (0, 0)),
        ],
        out_specs=pl.BlockSpec((1, n_cls), lambda i: (0, 0)),
        out_shape=jax.ShapeDtypeStruct((1, n_cls), jnp.float32),
        compiler_params=pltpu.CompilerParams(
            allow_input_fusion=[True, False, False, False, False]),
        scratch_shapes=[
            pltpu.VMEM((c_rows, c_rows), jnp.float32),
            pltpu.VMEM((c_rows, 8), jnp.float32),
            pltpu.VMEM((c_rows, 8), jnp.float32),
            pltpu.VMEM((c_rows, 8), jnp.float32),
        ],
    )(db, W_gfc, bg, W_cls, bc)
    return out


# speedup vs baseline: 1.0759x; 1.0611x over previous
import functools

import jax
import jax.numpy as jnp
from jax import lax
from jax.experimental import pallas as pl
from jax.experimental.pallas import tpu as pltpu

_THRESH = 0.6


def _fused_kernel(data_ref, wg_ref, bg_ref, wc_ref, bc_ref, out_ref,
                  g_ref, s1_ref, s34_ref, p_ref,
                  *, t_total, chunk, n_chunks):
    step = pl.program_id(0)

    @pl.when(step == 0)
    def _init():
        g_ref[...] = jnp.zeros_like(g_ref)
        s1_ref[...] = jnp.zeros_like(s1_ref)
        s34_ref[...] = jnp.zeros_like(s34_ref)

    c_rows = data_ref.shape[0]

    def _accum(cb):
        c2b = cb * cb
        p_ref[0:c_rows, :] = c2b * cb
        p_ref[c_rows:2 * c_rows, :] = c2b * c2b
        ones = jnp.ones((chunk, 8), dtype=jnp.bfloat16)
        g_ref[...] += lax.dot_general(
            cb, cb, (((1,), (1,)), ((), ())),
            preferred_element_type=jnp.float32)
        s34_ref[...] += jnp.dot(p_ref[...], ones,
                                preferred_element_type=jnp.float32)
        s1_ref[...] += jnp.sum(cb, axis=1, keepdims=True,
                               dtype=jnp.float32)

    @pl.when(step < n_chunks - 1)
    def _full_chunk():
        _accum(data_ref[...])

    @pl.when(step == n_chunks - 1)
    def _last_chunk():
        col = lax.broadcasted_iota(jnp.int32, (c_rows, chunk), 1)
        valid = col < (t_total - (n_chunks - 1) * chunk)
        _accum(jnp.where(valid, data_ref[...], jnp.bfloat16(0.0)))

    @pl.when(step == n_chunks - 1)
    def _finalize():
        t = jnp.float32(t_total)
        g = g_ref[...]
        row = lax.broadcasted_iota(jnp.int32, g.shape, 0)
        colid = lax.broadcasted_iota(jnp.int32, g.shape, 1)
        eye = row == colid

        s1 = s1_ref[...]
        s2 = jnp.sum(jnp.where(eye, g, 0.0), axis=1, keepdims=True)
        s3 = s34_ref[0:c_rows, 0:1]
        s4 = s34_ref[c_rows:2 * c_rows, 0:1]
        mu = s1 / t
        ex2 = s2 / t
        ex3 = s3 / t
        ex4 = s4 / t
        mu2 = mu * mu
        m2 = ex2 - mu2
        m3 = ex3 - 3.0 * mu * ex2 + 2.0 * mu * mu2
        m4 = ex4 - 4.0 * mu * ex3 + 6.0 * mu2 * ex2 - 3.0 * mu2 * mu2
        m2s = jnp.maximum(m2, 1e-12)
        skew = m3 / (m2s * jnp.sqrt(m2s))
        kurt = m4 / (m2s * m2s) - 3.0
        x = jnp.concatenate([mu, m2, skew, kurt], axis=1)

        outer = lax.dot_general(
            s1, s1, (((1,), (1,)), ((), ())),
            preferred_element_type=jnp.float32)
        cov = g - outer / t
        normsq = jnp.maximum(s2 - s1 * s1 / t, 0.0)
        norm = jnp.maximum(jnp.sqrt(normsq), 1e-6)
        denom = lax.dot_general(
            norm, norm, (((1,), (1,)), ((), ())),
            preferred_element_type=jnp.float32)
        corr = jnp.clip(cov / denom, -1.0, 1.0)

        acorr = jnp.abs(corr)
        w = jnp.clip(acorr, 1e-6, 0.99)
        a_mat = jnp.where(eye, 1.0, jnp.where(acorr >= _THRESH, w, 0.0))

        agg = jnp.dot(a_mat, x, preferred_element_type=jnp.float32)
        h = jnp.maximum(
            jnp.dot(agg, wg_ref[...], preferred_element_type=jnp.float32)
            + bg_ref[...], 0.0)
        emb = jnp.sum(h, axis=0, keepdims=True)
        logits = jnp.dot(emb, wc_ref[...],
                         preferred_element_type=jnp.float32) + bc_ref[...]
        out_ref[...] = logits


def kernel(data, W_gfc, b_gfc, W_cls, b_cls):
    c_rows, t_total = data.shape
    hidden = W_gfc.shape[1]
    n_cls = W_cls.shape[1]
    chunk = 2560
    n_chunks = (t_total + chunk - 1) // chunk

    bg = b_gfc.reshape(1, hidden)
    bc = b_cls.reshape(1, n_cls)
    db = data.astype(jnp.bfloat16)

    body = functools.partial(
        _fused_kernel, t_total=t_total, chunk=chunk, n_chunks=n_chunks)
    out = pl.pallas_call(
        body,
        grid=(n_chunks,),
        in_specs=[
            pl.BlockSpec((c_rows, chunk), lambda i: (0, i)),
            pl.BlockSpec(W_gfc.shape, lambda i: (0, 0)),
            pl.BlockSpec(bg.shape, lambda i: (0, 0)),
            pl.BlockSpec(W_cls.shape, lambda i: (0, 0)),
            pl.BlockSpec(bc.shape, lambda i: (0, 0)),
        ],
        out_specs=pl.BlockSpec((1, n_cls), lambda i: (0, 0)),
        out_shape=jax.ShapeDtypeStruct((1, n_cls), jnp.float32),
        compiler_params=pltpu.CompilerParams(
            allow_input_fusion=[True, False, False, False, False]),
        scratch_shapes=[
            pltpu.VMEM((c_rows, c_rows), jnp.float32),
            pltpu.VMEM((c_rows, 1), jnp.float32),
            pltpu.VMEM((2 * c_rows, 8), jnp.float32),
            pltpu.VMEM((2 * c_rows, chunk), jnp.bfloat16),
        ],
    )(db, W_gfc, bg, W_cls, bc)
    return out
